# Initial kernel scaffold; baseline (speedup 1.0000x reference)
import jax
import jax.numpy as jnp
from jax.experimental import pallas as pl


def kernel(z, edge_index, W1l, b1, W1r, W2l, b2, W2r, W3l, b3, W3r, W4l, b4, W4r):
    raise NotImplementedError("write your pallas kernel here")


# speedup vs baseline: 2.9136x; 2.9136x over previous
import functools

import jax
import jax.numpy as jnp
from jax import lax
from jax.experimental import pallas as pl
from jax.experimental.pallas import tpu as pltpu
from jax.experimental.pallas import tpu_sc as plsc

N_NODES = 10000
N_PAD = 10112
E_EDGES = 320000
E_PAD = 327680
NC = 2
NS = 16
RPT = N_PAD // NS
EPW = E_PAD // (NC * NS)
CHUNK = 256
KB = CHUNK // 128
NCHUNK = EPW // CHUNK


def _segsum_body(table, src2, dst2, zrows, zdeg, ones_h, out, deg_out,
                 src_idx, dst_idx, rows, acc, dega, ones_v, sem):
  c = lax.axis_index("c")
  s = lax.axis_index("s")
  pltpu.sync_copy(zrows, acc.at[pl.ds(s * RPT, RPT)])
  pltpu.sync_copy(zdeg, dega.at[pl.ds(s * RPT, RPT)])
  pltpu.sync_copy(ones_h, ones_v)
  plsc.subcore_barrier()

  w = s * NC + c
  row0 = w * (EPW // 128)

  def step(k, carry):
    base = row0 + k * KB
    pltpu.sync_copy(src2.at[pl.ds(base, KB)], src_idx)
    pltpu.sync_copy(dst2.at[pl.ds(base, KB)], dst_idx)
    cps = [
        pltpu.async_copy(table.at[src_idx.at[j]],
                         rows.at[pl.ds(j * 128, 128)], sem)
        for j in range(KB)
    ]
    for cp in cps:
      cp.wait()
    for j in range(KB):
      pltpu.sync_copy(rows.at[pl.ds(j * 128, 128)],
                      acc.at[dst_idx.at[j]], add=True)
      pltpu.sync_copy(ones_v, dega.at[dst_idx.at[j]], add=True)
    return carry

  lax.fori_loop(0, NCHUNK, step, 0)
  plsc.subcore_barrier()
  pltpu.sync_copy(acc.at[pl.ds(s * RPT, RPT)],
                  out.at[c].at[pl.ds(s * RPT, RPT)])
  pltpu.sync_copy(dega.at[pl.ds(s * RPT, RPT)],
                  deg_out.at[c].at[pl.ds(s * RPT, RPT)])


_segsum = pl.kernel(
    _segsum_body,
    out_type=(jax.ShapeDtypeStruct((NC, N_PAD, 128), jnp.float32),
              jax.ShapeDtypeStruct((NC, N_PAD, 16), jnp.float32)),
    mesh=plsc.VectorSubcoreMesh(core_axis_name="c", subcore_axis_name="s"),
    scratch_types=(
        pltpu.VMEM((KB, 128), jnp.int32),
        pltpu.VMEM((KB, 128), jnp.int32),
        pltpu.VMEM((CHUNK, 128), jnp.float32),
        pltpu.VMEM_SHARED((N_PAD, 128), jnp.float32),
        pltpu.VMEM_SHARED((N_PAD, 16), jnp.float32),
        pltpu.VMEM((128, 16), jnp.float32),
        pltpu.SemaphoreType.DMA,
    ),
    compiler_params=pltpu.CompilerParams(use_tc_tiling_on_sc=False),
)


_B = 1000


def _recip_deg(deg_ref):
  return 1.0 / jnp.maximum(deg_ref[0, :, 0:1] + deg_ref[1, :, 0:1], 1.0)


def _mean(s_ref, deg_ref):
  return (s_ref[0] + s_ref[1]) * _recip_deg(deg_ref)


def _tc1_body(s_ref, x_ref, deg_ref, wl_ref, b_ref, wr_ref, outa_ref,
              outb_ref):
  dot = functools.partial(jnp.dot, preferred_element_type=jnp.float32)
  h = (dot(_mean(s_ref, deg_ref), wl_ref[...]) + dot(x_ref[...], wr_ref[...])
       + b_ref[...])
  h = jnp.maximum(h, 0.0)
  outa_ref[...] = h[:, 0:128]
  outb_ref[...] = h[:, 128:256]


def _tc2_body(sa_ref, sb_ref, xa_ref, xb_ref, deg_ref, wl_ref, b_ref, wr_ref,
              w3l_ref, w3r_ref, p3_ref, r3_ref):
  dot = functools.partial(jnp.dot, preferred_element_type=jnp.float32)
  h = (dot(_mean(sa_ref, deg_ref), wl_ref[0:128, :])
       + dot(_mean(sb_ref, deg_ref), wl_ref[128:256, :])
       + dot(xa_ref[...], wr_ref[0:128, :])
       + dot(xb_ref[...], wr_ref[128:256, :])
       + b_ref[...])
  h = jnp.maximum(h, 0.0)
  p3_ref[...] = dot(h, w3l_ref[...])
  r3_ref[...] = dot(h, w3r_ref[...])


def _tc3_body(s_ref, r3_ref, deg_ref, b_ref, out_ref):
  out_ref[...] = jnp.maximum(_mean(s_ref, deg_ref) + r3_ref[...] + b_ref[...],
                             0.0)


def _tc4_body(s_ref, x_ref, deg_ref, wl_ref, b_ref, wr_ref, out_ref):
  dot = functools.partial(jnp.dot, preferred_element_type=jnp.float32)
  out_ref[...] = (dot(_mean(s_ref, deg_ref), wl_ref[...])
                  + dot(x_ref[...], wr_ref[...]) + b_ref[...])


_S_SPEC = pl.BlockSpec((2, _B, 128), lambda i: (0, i, 0))
_DEG_SPEC = pl.BlockSpec((2, _B, 16), lambda i: (0, i, 0))


def _x_spec(w):
  return pl.BlockSpec((_B, w), lambda i: (i, 0))


def _full_spec(shape):
  nd = len(shape)
  return pl.BlockSpec(shape, lambda i, _nd=nd: (0,) * _nd)


def _tc1(s1, x, deg, wl, b, wr):
  return pl.pallas_call(
      _tc1_body,
      grid=(N_NODES // _B,),
      in_specs=[_S_SPEC, _x_spec(128), _DEG_SPEC, _full_spec((128, 256)),
                _full_spec((1, 256)), _full_spec((128, 256))],
      out_specs=[_x_spec(128), _x_spec(128)],
      out_shape=[jax.ShapeDtypeStruct((N_NODES, 128), jnp.float32),
                 jax.ShapeDtypeStruct((N_NODES, 128), jnp.float32)],
  )(s1, x, deg, wl, b, wr)


def _tc2(s2a, s2b, xa, xb, deg, wl, b, wr, w3l, w3r):
  return pl.pallas_call(
      _tc2_body,
      grid=(N_NODES // _B,),
      in_specs=[_S_SPEC, _S_SPEC, _x_spec(128), _x_spec(128), _DEG_SPEC,
                _full_spec((256, 256)), _full_spec((1, 256)),
                _full_spec((256, 256)), _full_spec((256, 128)),
                _full_spec((256, 128))],
      out_specs=[_x_spec(128), _x_spec(128)],
      out_shape=[jax.ShapeDtypeStruct((N_NODES, 128), jnp.float32),
                 jax.ShapeDtypeStruct((N_NODES, 128), jnp.float32)],
  )(s2a, s2b, xa, xb, deg, wl, b, wr, w3l, w3r)


def _tc3(s3, r3, deg, b):
  return pl.pallas_call(
      _tc3_body,
      grid=(N_NODES // _B,),
      in_specs=[_S_SPEC, _x_spec(128), _DEG_SPEC, _full_spec((1, 128))],
      out_specs=_x_spec(128),
      out_shape=jax.ShapeDtypeStruct((N_NODES, 128), jnp.float32),
  )(s3, r3, deg, b)


def _tc4(s4, x, deg, wl, b, wr):
  return pl.pallas_call(
      _tc4_body,
      grid=(N_NODES // _B,),
      in_specs=[_S_SPEC, _x_spec(128), _DEG_SPEC, _full_spec((128, 128)),
                _full_spec((1, 128)), _full_spec((128, 128))],
      out_specs=_x_spec(128),
      out_shape=jax.ShapeDtypeStruct((N_NODES, 128), jnp.float32),
  )(s4, x, deg, wl, b, wr)


@jax.jit
def kernel(z, edge_index, W1l, b1, W1r, W2l, b2, W2r, W3l, b3, W3r, W4l, b4,
           W4r):
  src = edge_index[0].astype(jnp.int32)
  dst = edge_index[1].astype(jnp.int32)
  pad = E_PAD - E_EDGES
  src2 = jnp.concatenate([src, jnp.zeros((pad,), jnp.int32)]).reshape(-1, 128)
  dst2 = jnp.concatenate(
      [dst, jnp.full((pad,), N_NODES, jnp.int32)]).reshape(-1, 128)

  zrows = jnp.zeros((RPT, 128), jnp.float32)
  zdeg = jnp.zeros((RPT, 16), jnp.float32)
  ones_h = jnp.ones((128, 16), jnp.float32)
  b1r, b2r, b3r, b4r = (b.reshape(1, -1) for b in (b1, b2, b3, b4))

  s1, deg = _segsum(z, src2, dst2, zrows, zdeg, ones_h)
  h1a, h1b = _tc1(s1, z, deg, W1l, b1r, W1r)
  s2a, _ = _segsum(h1a, src2, dst2, zrows, zdeg, ones_h)
  s2b, _ = _segsum(h1b, src2, dst2, zrows, zdeg, ones_h)
  p3, r3 = _tc2(s2a, s2b, h1a, h1b, deg, W2l, b2r, W2r, W3l, W3r)
  s3, _ = _segsum(p3, src2, dst2, zrows, zdeg, ones_h)
  h3 = _tc3(s3, r3, deg, b3r)
  s4, _ = _segsum(h3, src2, dst2, zrows, zdeg, ones_h)
  return _tc4(s4, h3, deg, W4l, b4r, W4r)


# speedup vs baseline: 3.1187x; 1.0704x over previous
import functools

import jax
import jax.numpy as jnp
from jax import lax
from jax.experimental import pallas as pl
from jax.experimental.pallas import tpu as pltpu
from jax.experimental.pallas import tpu_sc as plsc

N_NODES = 10000
N_PAD = 10112
E_EDGES = 320000
E_PAD = 327680
NC = 2
NS = 16
RPT = N_PAD // NS
EPW = E_PAD // (NC * NS)
CHUNK = 128
NCHUNK = EPW // CHUNK
HC = NCHUNK // 2


def _segsum_body(table, src2, dst2, zrows, out,
                 sia, sib, dia, dib, rowsa, rowsb, acc, sema, semb):
  c = lax.axis_index("c")
  s = lax.axis_index("s")
  pltpu.sync_copy(zrows, acc.at[pl.ds(s * RPT, RPT)])
  plsc.subcore_barrier()

  w = s * NC + c
  row0 = w * (EPW // 128)

  def gather(si, rows, sem):
    return pltpu.async_copy(table.at[si.at[0]], rows, sem)

  pltpu.sync_copy(src2.at[pl.ds(row0, 1)], sia)
  pltpu.sync_copy(dst2.at[pl.ds(row0, 1)], dia)
  gather(sia, rowsa, sema)

  def step(k, carry):
    pltpu.sync_copy(src2.at[pl.ds(row0 + 2 * k + 1, 1)], sib)
    pltpu.sync_copy(dst2.at[pl.ds(row0 + 2 * k + 1, 1)], dib)
    gather(sib, rowsb, semb)
    pltpu.make_async_copy(table.at[sia.at[0]], rowsa, sema).wait()
    pltpu.sync_copy(rowsa, acc.at[dia.at[0]], add=True)
    @pl.when(k + 1 < HC)
    def _():
      pltpu.sync_copy(src2.at[pl.ds(row0 + 2 * k + 2, 1)], sia)
      pltpu.sync_copy(dst2.at[pl.ds(row0 + 2 * k + 2, 1)], dia)
      gather(sia, rowsa, sema)
    pltpu.make_async_copy(table.at[sib.at[0]], rowsb, semb).wait()
    pltpu.sync_copy(rowsb, acc.at[dib.at[0]], add=True)
    return carry

  lax.fori_loop(0, HC, step, 0)
  plsc.subcore_barrier()
  pltpu.sync_copy(acc.at[pl.ds(s * RPT, RPT)],
                  out.at[c].at[pl.ds(s * RPT, RPT)])


_segsum = pl.kernel(
    _segsum_body,
    out_type=(jax.ShapeDtypeStruct((NC, N_PAD, 128), jnp.float32),),
    mesh=plsc.VectorSubcoreMesh(core_axis_name="c", subcore_axis_name="s"),
    scratch_types=(
        pltpu.VMEM((1, 128), jnp.int32),
        pltpu.VMEM((1, 128), jnp.int32),
        pltpu.VMEM((1, 128), jnp.int32),
        pltpu.VMEM((1, 128), jnp.int32),
        pltpu.VMEM((CHUNK, 128), jnp.float32),
        pltpu.VMEM((CHUNK, 128), jnp.float32),
        pltpu.VMEM_SHARED((N_PAD, 128), jnp.float32),
        pltpu.SemaphoreType.DMA,
        pltpu.SemaphoreType.DMA,
    ),
    compiler_params=pltpu.CompilerParams(use_tc_tiling_on_sc=False),
)


def _deg_body(dst2, zdeg, ones_h, deg_out, didx, dega, ones_v, sem):
  c = lax.axis_index("c")
  s = lax.axis_index("s")
  pltpu.sync_copy(zdeg, dega.at[pl.ds(s * RPT, RPT)])
  pltpu.sync_copy(ones_h, ones_v)
  plsc.subcore_barrier()

  w = s * NC + c
  row0 = w * (EPW // 128)

  def step(k, carry):
    pltpu.sync_copy(dst2.at[pl.ds(row0 + 8 * k, 8)], didx)
    for j in range(8):
      pltpu.sync_copy(ones_v, dega.at[didx.at[j]], add=True)
    return carry

  lax.fori_loop(0, (EPW // 128) // 8, step, 0)
  plsc.subcore_barrier()
  pltpu.sync_copy(dega.at[pl.ds(s * RPT, RPT)],
                  deg_out.at[c].at[pl.ds(s * RPT, RPT)])
  del sem


_deg = pl.kernel(
    _deg_body,
    out_type=(jax.ShapeDtypeStruct((NC, N_PAD, 16), jnp.float32),),
    mesh=plsc.VectorSubcoreMesh(core_axis_name="c", subcore_axis_name="s"),
    scratch_types=(
        pltpu.VMEM((8, 128), jnp.int32),
        pltpu.VMEM_SHARED((N_PAD, 16), jnp.float32),
        pltpu.VMEM((128, 16), jnp.float32),
        pltpu.SemaphoreType.DMA,
    ),
    compiler_params=pltpu.CompilerParams(use_tc_tiling_on_sc=False),
)


_B = 1000


def _recip_deg(deg_ref):
  return 1.0 / jnp.maximum(deg_ref[0, :, 0:1] + deg_ref[1, :, 0:1], 1.0)


def _mean(s_ref, deg_ref):
  return (s_ref[0] + s_ref[1]) * _recip_deg(deg_ref)


def _tc1_body(s_ref, x_ref, deg_ref, wl_ref, b_ref, wr_ref, outa_ref,
              outb_ref):
  dot = functools.partial(jnp.dot, preferred_element_type=jnp.float32)
  h = (dot(_mean(s_ref, deg_ref), wl_ref[...]) + dot(x_ref[...], wr_ref[...])
       + b_ref[...])
  h = jnp.maximum(h, 0.0)
  outa_ref[...] = h[:, 0:128]
  outb_ref[...] = h[:, 128:256]


def _tc2_body(sa_ref, sb_ref, xa_ref, xb_ref, deg_ref, wl_ref, b_ref, wr_ref,
              w3l_ref, w3r_ref, p3_ref, r3_ref):
  dot = functools.partial(jnp.dot, preferred_element_type=jnp.float32)
  h = (dot(_mean(sa_ref, deg_ref), wl_ref[0:128, :])
       + dot(_mean(sb_ref, deg_ref), wl_ref[128:256, :])
       + dot(xa_ref[...], wr_ref[0:128, :])
       + dot(xb_ref[...], wr_ref[128:256, :])
       + b_ref[...])
  h = jnp.maximum(h, 0.0)
  p3_ref[...] = dot(h, w3l_ref[...])
  r3_ref[...] = dot(h, w3r_ref[...])


def _tc3_body(s_ref, r3_ref, deg_ref, b_ref, out_ref):
  out_ref[...] = jnp.maximum(_mean(s_ref, deg_ref) + r3_ref[...] + b_ref[...],
                             0.0)


def _tc4_body(s_ref, x_ref, deg_ref, wl_ref, b_ref, wr_ref, out_ref):
  dot = functools.partial(jnp.dot, preferred_element_type=jnp.float32)
  out_ref[...] = (dot(_mean(s_ref, deg_ref), wl_ref[...])
                  + dot(x_ref[...], wr_ref[...]) + b_ref[...])


_S_SPEC = pl.BlockSpec((2, _B, 128), lambda i: (0, i, 0))
_DEG_SPEC = pl.BlockSpec((2, _B, 16), lambda i: (0, i, 0))


def _x_spec(w):
  return pl.BlockSpec((_B, w), lambda i: (i, 0))


def _full_spec(shape):
  nd = len(shape)
  return pl.BlockSpec(shape, lambda i, _nd=nd: (0,) * _nd)


def _tc1(s1, x, deg, wl, b, wr):
  return pl.pallas_call(
      _tc1_body,
      grid=(N_NODES // _B,),
      in_specs=[_S_SPEC, _x_spec(128), _DEG_SPEC, _full_spec((128, 256)),
                _full_spec((1, 256)), _full_spec((128, 256))],
      out_specs=[_x_spec(128), _x_spec(128)],
      out_shape=[jax.ShapeDtypeStruct((N_NODES, 128), jnp.float32),
                 jax.ShapeDtypeStruct((N_NODES, 128), jnp.float32)],
  )(s1, x, deg, wl, b, wr)


def _tc2(s2a, s2b, xa, xb, deg, wl, b, wr, w3l, w3r):
  return pl.pallas_call(
      _tc2_body,
      grid=(N_NODES // _B,),
      in_specs=[_S_SPEC, _S_SPEC, _x_spec(128), _x_spec(128), _DEG_SPEC,
                _full_spec((256, 256)), _full_spec((1, 256)),
                _full_spec((256, 256)), _full_spec((256, 128)),
                _full_spec((256, 128))],
      out_specs=[_x_spec(128), _x_spec(128)],
      out_shape=[jax.ShapeDtypeStruct((N_NODES, 128), jnp.float32),
                 jax.ShapeDtypeStruct((N_NODES, 128), jnp.float32)],
  )(s2a, s2b, xa, xb, deg, wl, b, wr, w3l, w3r)


def _tc3(s3, r3, deg, b):
  return pl.pallas_call(
      _tc3_body,
      grid=(N_NODES // _B,),
      in_specs=[_S_SPEC, _x_spec(128), _DEG_SPEC, _full_spec((1, 128))],
      out_specs=_x_spec(128),
      out_shape=jax.ShapeDtypeStruct((N_NODES, 128), jnp.float32),
  )(s3, r3, deg, b)


def _tc4(s4, x, deg, wl, b, wr):
  return pl.pallas_call(
      _tc4_body,
      grid=(N_NODES // _B,),
      in_specs=[_S_SPEC, _x_spec(128), _DEG_SPEC, _full_spec((128, 128)),
                _full_spec((1, 128)), _full_spec((128, 128))],
      out_specs=_x_spec(128),
      out_shape=jax.ShapeDtypeStruct((N_NODES, 128), jnp.float32),
  )(s4, x, deg, wl, b, wr)


@jax.jit
def kernel(z, edge_index, W1l, b1, W1r, W2l, b2, W2r, W3l, b3, W3r, W4l, b4,
           W4r):
  src = edge_index[0].astype(jnp.int32)
  dst = edge_index[1].astype(jnp.int32)
  pad = E_PAD - E_EDGES
  src2 = jnp.concatenate([src, jnp.zeros((pad,), jnp.int32)]).reshape(-1, 128)
  dst2 = jnp.concatenate(
      [dst, jnp.full((pad,), N_NODES, jnp.int32)]).reshape(-1, 128)

  zrows = jnp.zeros((RPT, 128), jnp.float32)
  zdeg = jnp.zeros((RPT, 16), jnp.float32)
  ones_h = jnp.ones((128, 16), jnp.float32)
  b1r, b2r, b3r, b4r = (b.reshape(1, -1) for b in (b1, b2, b3, b4))

  (deg,) = _deg(dst2, zdeg, ones_h)
  (s1,) = _segsum(z, src2, dst2, zrows)
  h1a, h1b = _tc1(s1, z, deg, W1l, b1r, W1r)
  (s2a,) = _segsum(h1a, src2, dst2, zrows)
  (s2b,) = _segsum(h1b, src2, dst2, zrows)
  p3, r3 = _tc2(s2a, s2b, h1a, h1b, deg, W2l, b2r, W2r, W3l, W3r)
  (s3,) = _segsum(p3, src2, dst2, zrows)
  h3 = _tc3(s3, r3, deg, b3r)
  (s4,) = _segsum(h3, src2, dst2, zrows)
  return _tc4(s4, h3, deg, W4l, b4r, W4r)
